# baseline (speedup 1.0000x reference)
import functools

import jax
import jax.numpy as jnp
from jax import lax
from jax.experimental import pallas as pl
from jax.experimental.pallas import tpu as pltpu
from jax.experimental.pallas import tpu_sc as plsc

_R = 8
_W = 128
_NBUF = 3


def kernel(x, emb):
    batch, seq_len, model_dim = x.shape
    info = plsc.get_sparse_core_info()
    nc, ns = info.num_cores, info.num_subcores
    nw = nc * ns
    rows_pw = seq_len // nw
    n_chunks = rows_pw // _R
    cr = _R * model_dim // _W
    x4 = x.reshape(-1, _W)
    emb4 = emb.reshape(-1, _W)
    b_stride = seq_len * model_dim // _W
    mesh = plsc.VectorSubcoreMesh(core_axis_name="c", subcore_axis_name="s")

    @functools.partial(
        pl.kernel,
        out_type=jax.ShapeDtypeStruct(x4.shape, x.dtype),
        mesh=mesh,
        scratch_types=[
            [pltpu.VMEM((cr, _W), jnp.float32) for _ in range(_NBUF)],
            [pltpu.VMEM((cr,), jnp.int32) for _ in range(2)],
            [pltpu.SemaphoreType.DMA for _ in range(3 * _NBUF)],
        ],
    )
    def sc_add(x_hbm, emb_hbm, out_hbm, xbufs, idxvs, sems):
        cid = lax.axis_index("c")
        sid = lax.axis_index("s")
        wid = sid * nc + cid
        in_sems = sems[:_NBUF]
        out_sems = sems[_NBUF:2 * _NBUF]
        add_sems = sems[2 * _NBUF:]

        stages = [(g, b) for g in range(n_chunks) for b in range(batch)]

        def write_idx(g):
            iv = idxvs[g % 2]
            for j in range(cr // 16):
                iv[pl.ds(j * 16, 16)] = lax.iota(jnp.int32, 16) + (
                    (wid * n_chunks + g) * cr + j * 16
                )

        def start_in(k):
            g, b = stages[k]
            r0 = b * b_stride + (wid * n_chunks + g) * cr
            return pltpu.async_copy(
                x_hbm.at[pl.ds(r0, cr), :], xbufs[k % _NBUF], in_sems[k % _NBUF]
            )

        def start_out(k):
            g, b = stages[k]
            r0 = b * b_stride + (wid * n_chunks + g) * cr
            return pltpu.async_copy(
                xbufs[k % _NBUF], out_hbm.at[pl.ds(r0, cr), :], out_sems[k % _NBUF]
            )

        def start_add(k):
            g, _ = stages[k]
            return pltpu.async_copy(
                emb_hbm.at[idxvs[g % 2]], xbufs[k % _NBUF], add_sems[k % _NBUF],
                add=True,
            )

        in_descs = [None] * len(stages)
        out_descs = [None] * len(stages)
        add_descs = [None] * len(stages)
        write_idx(0)
        in_descs[0] = start_in(0)
        for k, (g, b) in enumerate(stages):
            if k > 0:
                add_descs[k - 1].wait()
                out_descs[k - 1] = start_out(k - 1)
            if b == 0 and g > 0:
                write_idx(g)
            in_descs[k].wait()
            add_descs[k] = start_add(k)
            if k + 1 < len(stages):
                if out_descs[k + 1 - _NBUF] is not None:
                    out_descs[k + 1 - _NBUF].wait()
                in_descs[k + 1] = start_in(k + 1)
        last = len(stages) - 1
        add_descs[last].wait()
        out_descs[last] = start_out(last)
        for k in range(len(stages) - _NBUF, len(stages)):
            out_descs[k].wait()

    out4 = sc_add(x4, emb4)
    return out4.reshape(batch, seq_len, model_dim)


# speedup vs baseline: 1.0828x; 1.0828x over previous
import functools

import jax
import jax.numpy as jnp
from jax import lax
from jax.experimental import pallas as pl
from jax.experimental.pallas import tpu as pltpu
from jax.experimental.pallas import tpu_sc as plsc

_R = 8
_NBUF = 3


def kernel(x, emb):
    batch, seq_len, model_dim = x.shape
    info = plsc.get_sparse_core_info()
    nc, ns = info.num_cores, info.num_subcores
    nw = nc * ns
    rows_pw = seq_len // nw
    n_chunks = rows_pw // _R
    x2 = x.reshape(-1, model_dim)
    pos = jnp.arange(seq_len, dtype=jnp.int32).reshape(-1, _R)
    mesh = plsc.VectorSubcoreMesh(core_axis_name="c", subcore_axis_name="s")

    @functools.partial(
        pl.kernel,
        out_type=jax.ShapeDtypeStruct(x2.shape, x.dtype),
        mesh=mesh,
        scratch_types=[
            [pltpu.VMEM((_R, model_dim), jnp.float32) for _ in range(_NBUF)],
            pltpu.VMEM((rows_pw // _R, _R), jnp.int32),
            [pltpu.SemaphoreType.DMA for _ in range(3 * _NBUF)],
        ],
        compiler_params=pltpu.CompilerParams(use_tc_tiling_on_sc=False),
    )
    def sc_add(x_hbm, emb_hbm, pos_hbm, out_hbm, xbufs, idxv, sems):
        cid = lax.axis_index("c")
        sid = lax.axis_index("s")
        wid = sid * nc + cid
        s_base = wid * rows_pw
        in_sems = sems[:_NBUF]
        out_sems = sems[_NBUF:2 * _NBUF]
        add_sems = sems[2 * _NBUF:]

        pltpu.sync_copy(pos_hbm.at[pl.ds(wid * n_chunks, n_chunks), :], idxv)

        stages = [(g, b) for g in range(n_chunks) for b in range(batch)]

        def start_in(k):
            g, b = stages[k]
            r0 = b * seq_len + s_base + g * _R
            return pltpu.async_copy(
                x_hbm.at[pl.ds(r0, _R), :], xbufs[k % _NBUF], in_sems[k % _NBUF]
            )

        def start_out(k):
            g, b = stages[k]
            r0 = b * seq_len + s_base + g * _R
            return pltpu.async_copy(
                xbufs[k % _NBUF], out_hbm.at[pl.ds(r0, _R), :], out_sems[k % _NBUF]
            )

        def start_add(k):
            g, _ = stages[k]
            return pltpu.async_copy(
                emb_hbm.at[idxv.at[g]], xbufs[k % _NBUF],
                add_sems[k % _NBUF], add=True,
            )

        in_descs = [None] * len(stages)
        out_descs = [None] * len(stages)
        add_descs = [None] * len(stages)
        in_descs[0] = start_in(0)
        for k, (g, b) in enumerate(stages):
            if k > 0:
                add_descs[k - 1].wait()
                out_descs[k - 1] = start_out(k - 1)
            in_descs[k].wait()
            add_descs[k] = start_add(k)
            if k + 1 < len(stages):
                if out_descs[k + 1 - _NBUF] is not None:
                    out_descs[k + 1 - _NBUF].wait()
                in_descs[k + 1] = start_in(k + 1)
        last = len(stages) - 1
        add_descs[last].wait()
        out_descs[last] = start_out(last)
        for k in range(len(stages) - _NBUF, len(stages)):
            out_descs[k].wait()

    out2 = sc_add(x2, emb, pos)
    return out2.reshape(batch, seq_len, model_dim)


# speedup vs baseline: 1.1496x; 1.0617x over previous
import functools

import jax
import jax.numpy as jnp
from jax import lax
from jax.experimental import pallas as pl
from jax.experimental.pallas import tpu as pltpu
from jax.experimental.pallas import tpu_sc as plsc

_R = 16
_NBUF = 2


def kernel(x, emb):
    batch, seq_len, model_dim = x.shape
    info = plsc.get_sparse_core_info()
    nc, ns = info.num_cores, info.num_subcores
    nw = nc * ns
    rows_pw = seq_len // nw
    n_chunks = rows_pw // _R
    x2 = x.reshape(-1, model_dim)
    pos = jnp.arange(seq_len, dtype=jnp.int32).reshape(-1, _R)
    mesh = plsc.VectorSubcoreMesh(core_axis_name="c", subcore_axis_name="s")

    @functools.partial(
        pl.kernel,
        out_type=jax.ShapeDtypeStruct(x2.shape, x.dtype),
        mesh=mesh,
        scratch_types=[
            [pltpu.VMEM((_R, model_dim), jnp.float32) for _ in range(_NBUF)],
            pltpu.VMEM((rows_pw // _R, _R), jnp.int32),
            [pltpu.SemaphoreType.DMA for _ in range(3 * _NBUF)],
        ],
        compiler_params=pltpu.CompilerParams(use_tc_tiling_on_sc=False),
    )
    def sc_add(x_hbm, emb_hbm, pos_hbm, out_hbm, xbufs, idxv, sems):
        cid = lax.axis_index("c")
        sid = lax.axis_index("s")
        wid = sid * nc + cid
        s_base = wid * rows_pw
        in_sems = sems[:_NBUF]
        out_sems = sems[_NBUF:2 * _NBUF]
        add_sems = sems[2 * _NBUF:]

        pltpu.sync_copy(pos_hbm.at[pl.ds(wid * n_chunks, n_chunks), :], idxv)

        stages = [(g, b) for g in range(n_chunks) for b in range(batch)]

        def start_in(k):
            g, b = stages[k]
            r0 = b * seq_len + s_base + g * _R
            return pltpu.async_copy(
                x_hbm.at[pl.ds(r0, _R), :], xbufs[k % _NBUF], in_sems[k % _NBUF]
            )

        def start_out(k):
            g, b = stages[k]
            r0 = b * seq_len + s_base + g * _R
            return pltpu.async_copy(
                xbufs[k % _NBUF], out_hbm.at[pl.ds(r0, _R), :], out_sems[k % _NBUF]
            )

        def start_add(k):
            g, _ = stages[k]
            return pltpu.async_copy(
                emb_hbm.at[idxv.at[g]], xbufs[k % _NBUF],
                add_sems[k % _NBUF], add=True,
            )

        in_descs = [None] * len(stages)
        out_descs = [None] * len(stages)
        add_descs = [None] * len(stages)
        in_descs[0] = start_in(0)
        for k, (g, b) in enumerate(stages):
            if k > 0:
                add_descs[k - 1].wait()
                out_descs[k - 1] = start_out(k - 1)
            in_descs[k].wait()
            add_descs[k] = start_add(k)
            if k + 1 < len(stages):
                if out_descs[k + 1 - _NBUF] is not None:
                    out_descs[k + 1 - _NBUF].wait()
                in_descs[k + 1] = start_in(k + 1)
        last = len(stages) - 1
        add_descs[last].wait()
        out_descs[last] = start_out(last)
        for k in range(len(stages) - _NBUF, len(stages)):
            out_descs[k].wait()

    out2 = sc_add(x2, emb, pos)
    return out2.reshape(batch, seq_len, model_dim)


# speedup vs baseline: 5.1475x; 4.4774x over previous
import jax
import jax.numpy as jnp
from jax.experimental import pallas as pl

_BLOCK_S = 1024


def _add_kernel(x_ref, emb_ref, out_ref):
    out_ref[...] = x_ref[...] + emb_ref[...]


def kernel(x, emb):
    batch, seq_len, model_dim = x.shape
    bs = _BLOCK_S
    n_s = seq_len // bs
    x2 = x.reshape(batch * seq_len, model_dim)
    out2 = pl.pallas_call(
        _add_kernel,
        grid=(n_s, batch),
        in_specs=[
            pl.BlockSpec((bs, model_dim), lambda s, b, n_s=n_s: (b * n_s + s, 0)),
            pl.BlockSpec((bs, model_dim), lambda s, b: (s, 0)),
        ],
        out_specs=pl.BlockSpec((bs, model_dim), lambda s, b, n_s=n_s: (b * n_s + s, 0)),
        out_shape=jax.ShapeDtypeStruct(x2.shape, x.dtype),
    )(x2, emb)
    return out2.reshape(batch, seq_len, model_dim)


# speedup vs baseline: 5.3454x; 1.0385x over previous
import jax
import jax.numpy as jnp
from jax.experimental import pallas as pl

_BLOCK_S = 2048


def _add_kernel(x_ref, emb_ref, out_ref):
    out_ref[...] = x_ref[...] + emb_ref[...]


def kernel(x, emb):
    batch, seq_len, model_dim = x.shape
    bs = _BLOCK_S
    n_s = seq_len // bs
    x2 = x.reshape(batch * seq_len, model_dim)
    out2 = pl.pallas_call(
        _add_kernel,
        grid=(n_s, batch),
        in_specs=[
            pl.BlockSpec((bs, model_dim), lambda s, b, n_s=n_s: (b * n_s + s, 0)),
            pl.BlockSpec((bs, model_dim), lambda s, b: (s, 0)),
        ],
        out_specs=pl.BlockSpec((bs, model_dim), lambda s, b, n_s=n_s: (b * n_s + s, 0)),
        out_shape=jax.ShapeDtypeStruct(x2.shape, x.dtype),
    )(x2, emb)
    return out2.reshape(batch, seq_len, model_dim)
